# Initial kernel scaffold; baseline (speedup 1.0000x reference)
import jax
import jax.numpy as jnp
from jax.experimental import pallas as pl


def kernel(src, emb_weight):
    raise NotImplementedError("write your pallas kernel here")


# speedup vs baseline: 5.6009x; 5.6009x over previous
import functools

import jax
import jax.numpy as jnp
from jax import lax
from jax.experimental import pallas as pl
from jax.experimental.pallas import tpu as pltpu
from jax.experimental.pallas import tpu_sc as plsc

NC, NS = 2, 16
NW = NC * NS
BATCH, HIST, D = 4096, 200, 128
B = BATCH * HIST
RPW = B // NW
G = 128
NG = RPW // G


@jax.jit
def _sc_gather(src_flat, emb_weight):
  mesh = plsc.VectorSubcoreMesh(
      core_axis_name="c", subcore_axis_name="s",
      num_cores=NC, num_subcores=NS)

  @functools.partial(
      pl.kernel,
      out_type=jax.ShapeDtypeStruct((NW * NG, G, D), jnp.float32),
      mesh=mesh,
      scratch_types=[
          pltpu.VMEM((NG, G), jnp.int32),
          pltpu.VMEM((2, G, D), jnp.float32),
          pltpu.SemaphoreType.DMA,
          pltpu.SemaphoreType.DMA,
      ],
  )
  def k(idx_hbm, table_hbm, out_hbm, idx_v, rows_v, gsem, osem):
    wid = lax.axis_index("s") * NC + lax.axis_index("c")
    pltpu.sync_copy(idx_hbm.at[wid], idx_v)
    obase = wid * NG

    def body(j, _):
      pltpu.async_copy(table_hbm.at[idx_v.at[j]], rows_v.at[0], gsem).wait()
      pltpu.sync_copy(rows_v.at[0], out_hbm.at[obase + j])
      return 0

    lax.fori_loop(0, NG, body, 0)

  return k(src_flat, emb_weight)


def kernel(src, emb_weight):
  src_flat = src.reshape(NW, NG, G)
  out = _sc_gather(src_flat, emb_weight)
  return out.reshape(BATCH, HIST, D)


# speedup vs baseline: 6.8074x; 1.2154x over previous
import functools

import jax
import jax.numpy as jnp
from jax import lax
from jax.experimental import pallas as pl
from jax.experimental.pallas import tpu as pltpu
from jax.experimental.pallas import tpu_sc as plsc

NC, NS = 2, 16
NW = NC * NS
BATCH, HIST, D = 4096, 200, 128
B = BATCH * HIST
RPW = B // NW
G = 128
NG = RPW // G


@jax.jit
def _sc_gather(src_flat, emb_weight):
  mesh = plsc.VectorSubcoreMesh(
      core_axis_name="c", subcore_axis_name="s",
      num_cores=NC, num_subcores=NS)

  @functools.partial(
      pl.kernel,
      out_type=jax.ShapeDtypeStruct((NW * NG, G, D), jnp.float32),
      mesh=mesh,
      scratch_types=[
          pltpu.VMEM((NG, G), jnp.int32),
          pltpu.VMEM((2, G, D), jnp.float32),
          pltpu.SemaphoreType.DMA,
          pltpu.SemaphoreType.DMA,
      ],
  )
  def k(idx_hbm, table_hbm, out_hbm, idx_v, rows_v, sem0, sem1):
    wid = lax.axis_index("s") * NC + lax.axis_index("c")
    pltpu.sync_copy(idx_hbm.at[wid], idx_v)
    obase = wid * NG
    sems = (sem0, sem1)

    for b in range(2):
      pltpu.async_copy(table_hbm.at[idx_v.at[b]], rows_v.at[b], sems[b])

    def body(jj, _):
      for b in range(2):
        j = jj + b
        pltpu.make_async_copy(
            table_hbm.at[idx_v.at[j]], rows_v.at[b], sems[b]).wait()
        pltpu.sync_copy(rows_v.at[b], out_hbm.at[obase + j])

        @pl.when(j + 2 < NG)
        def _():
          pltpu.async_copy(
              table_hbm.at[idx_v.at[j + 2]], rows_v.at[b], sems[b])
      return 0

    lax.fori_loop(0, NG // 2, lambda i, c: body(i * 2, c), 0)

  return k(src_flat, emb_weight)


def kernel(src, emb_weight):
  src_flat = src.reshape(NW, NG, G)
  out = _sc_gather(src_flat, emb_weight)
  return out.reshape(BATCH, HIST, D)


# speedup vs baseline: 15.6947x; 2.3055x over previous
import functools

import jax
import jax.numpy as jnp
from jax import lax
from jax.experimental import pallas as pl
from jax.experimental.pallas import tpu as pltpu
from jax.experimental.pallas import tpu_sc as plsc

NC, NS = 2, 16
NW = NC * NS
BATCH, HIST, D = 4096, 200, 128
VOCAB = 1000
B = BATCH * HIST
RPW = B // NW
G = 128
NG = RPW // G


@jax.jit
def _sc_gather(src_flat, emb_weight):
  mesh = plsc.VectorSubcoreMesh(
      core_axis_name="c", subcore_axis_name="s",
      num_cores=NC, num_subcores=NS)

  @functools.partial(
      pl.kernel,
      out_type=jax.ShapeDtypeStruct((NW * NG, G, D), jnp.float32),
      mesh=mesh,
      scratch_types=[
          pltpu.VMEM((NG, G), jnp.int32),
          pltpu.VMEM((2, G, D), jnp.float32),
          pltpu.SemaphoreType.DMA,
          pltpu.SemaphoreType.DMA,
          pltpu.VMEM_SHARED((VOCAB, D), jnp.float32),
      ],
  )
  def k(idx_hbm, table_hbm, out_hbm, idx_v, rows_v, sem0, sem1, table_sh):
    wid = lax.axis_index("s") * NC + lax.axis_index("c")

    @pl.when(lax.axis_index("s") == 0)
    def _():
      pltpu.sync_copy(table_hbm, table_sh)

    pltpu.sync_copy(idx_hbm.at[wid], idx_v)
    plsc.subcore_barrier()

    obase = wid * NG
    sems = (sem0, sem1)

    for b in range(2):
      pltpu.async_copy(table_sh.at[idx_v.at[b]], rows_v.at[b], sems[b])

    def body(jj, _):
      for b in range(2):
        j = jj + b
        pltpu.make_async_copy(
            table_sh.at[idx_v.at[j]], rows_v.at[b], sems[b]).wait()
        pltpu.sync_copy(rows_v.at[b], out_hbm.at[obase + j])

        @pl.when(j + 2 < NG)
        def _():
          pltpu.async_copy(
              table_sh.at[idx_v.at[j + 2]], rows_v.at[b], sems[b])
      return 0

    lax.fori_loop(0, NG // 2, lambda i, c: body(i * 2, c), 0)

  return k(src_flat, emb_weight)


def kernel(src, emb_weight):
  src_flat = src.reshape(NW, NG, G)
  out = _sc_gather(src_flat, emb_weight)
  return out.reshape(BATCH, HIST, D)
